# baseline (speedup 1.0000x reference)
import functools

import jax
import jax.numpy as jnp
import numpy as np
from jax import lax
from jax.experimental import pallas as pl
from jax.experimental.pallas import tpu as pltpu
from jax.experimental.pallas import tpu_sc as plsc

D = 128
N_SEG = 500000
SEG_TILE = 400
N_TILES = N_SEG // SEG_TILE
CHUNK = 128
REGION = 416
ZROWS = REGION // 4


def _mlp_body(s_ref, w1_ref, wd_ref, w2_ref, b_ref, o_ref):
  s = s_ref[...].astype(jnp.bfloat16)
  h = lax.dot_general(s, w1_ref[...].astype(jnp.bfloat16),
                      (((1,), (1,)), ((), ())),
                      preferred_element_type=jnp.float32)
  h = jnp.maximum(h, 0.0).astype(jnp.bfloat16)
  r = lax.dot_general(w2_ref[...].astype(jnp.bfloat16), h,
                      (((1,), (1,)), ((), ())),
                      preferred_element_type=jnp.float32)
  d = lax.dot_general(wd_ref[...].astype(jnp.bfloat16), s,
                      (((1,), (1,)), ((), ())),
                      preferred_element_type=jnp.float32)
  o_ref[...] = (r + d + b_ref[...])[None]


def kernel(fragment_embedding, fragment_cellxgene_ix, cell_n, gene_n, gene_ix,
           W_direct, W1, W2, bias1):
  gene_n_static = gene_ix.shape[0]
  cell_n_static = 5000
  zero_fold = jnp.asarray(cell_n * gene_n - cell_n_static * gene_n_static,
                          dtype=fragment_cellxgene_ix.dtype)
  ix = fragment_cellxgene_ix + zero_fold

  tile_starts = jnp.arange(N_TILES + 1, dtype=jnp.int32) * SEG_TILE
  bnd = jnp.searchsorted(ix, tile_starts, side="left")
  pad = -(N_TILES + 1) % 16 + 16
  bnd = jnp.concatenate(
      [bnd.astype(jnp.int32), jnp.zeros((pad,), jnp.int32)])

  mesh = plsc.VectorSubcoreMesh(core_axis_name="c", subcore_axis_name="s")
  info = plsc.get_sparse_core_info()
  nw = info.num_cores * info.num_subcores

  NPART = 10
  PART_TILES = N_TILES // NPART
  PART_SEGS = PART_TILES * SEG_TILE

  def make_sc_pool(tile0):
    @functools.partial(
        pl.kernel,
        out_type=jax.ShapeDtypeStruct((PART_SEGS, D), jnp.float32),
        mesh=mesh,
        scratch_types=[
            pltpu.VMEM((2, CHUNK, D), jnp.float32),
            pltpu.VMEM((CHUNK,), jnp.int32),
            pltpu.VMEM((2, CHUNK), jnp.int32),
            pltpu.VMEM((ZROWS, D), jnp.float32),
            pltpu.VMEM((N_TILES + 1 + (-(N_TILES + 1) % 16) + 16,), jnp.int32),
            pltpu.VMEM_SHARED((16 * REGION, D), jnp.float32),
            pltpu.SemaphoreType.DMA,
            pltpu.SemaphoreType.DMA,
            pltpu.SemaphoreType.DMA,
        ],
        name=f"sc_pool_{tile0}",
    )
    def sc_pool(x_hbm, ix_hbm, bnd_hbm, out_hbm, rowbuf, idx_loc, idx_raw,
                zerobuf, bndbuf, shared, sem_z, sem_b0, sem_b1):
      cid = lax.axis_index("c")
      sid = lax.axis_index("s")
      wid = cid * info.num_subcores + sid
      max_tiles = -(-PART_TILES // nw)
      region0 = sid * REGION

      def _zero_row(r, carry):
        for j in range(D // 16):
          zerobuf[r, pl.ds(j * 16, 16)] = jnp.zeros((16,), jnp.float32)
        return carry
      lax.fori_loop(0, ZROWS, _zero_row, 0)

      pltpu.sync_copy(bnd_hbm, bndbuf)

      def read_bnd(i):
        base = lax.div(i, 16) * 16
        off = i - base
        v = bndbuf[pl.ds(base, 16)]
        res = v[0]
        for k in range(1, 16):
          res = jnp.where(off == k, v[k], res)
        return res

      sems = (sem_b0, sem_b1)

      def issue_in(c, b):
        fs = c * CHUNK
        pltpu.async_copy(x_hbm.at[pl.ds(fs, CHUNK), :], rowbuf.at[b], sems[b])
        pltpu.async_copy(ix_hbm.at[pl.ds(fs, CHUNK)], idx_raw.at[b], sems[b])

      def wait_in(c, b):
        fs = c * CHUNK
        pltpu.make_async_copy(
            x_hbm.at[pl.ds(fs, CHUNK), :], rowbuf.at[b], sems[b]).wait()
        pltpu.make_async_copy(
            ix_hbm.at[pl.ds(fs, CHUNK)], idx_raw.at[b], sems[b]).wait()

      def do_tile(t, carry):
        gl = t * nw + wid

        @pl.when(gl < PART_TILES)
        def _():
          g = tile0 + gl
          seg_base = g * SEG_TILE
          out_base = gl * SEG_TILE
          f0 = read_bnd(g)
          f1 = read_bnd(g + 1)
          c0 = lax.div(f0, CHUNK)
          c1 = jnp.where(f1 > f0, lax.div(f1 + (CHUNK - 1), CHUNK), c0)

          for z in range(REGION // ZROWS):
            pltpu.async_copy(zerobuf,
                             shared.at[pl.ds(region0 + z * ZROWS, ZROWS)],
                             sem_z)
          for b in range(2):
            @pl.when(c0 + b < c1)
            def _issue(b=b):
              issue_in(c0 + b, b)
          for z in range(REGION // ZROWS):
            pltpu.make_async_copy(
                zerobuf, shared.at[pl.ds(region0 + z * ZROWS, ZROWS)],
                sem_z).wait()

          def do_pair(p, inner):
            for b in range(2):
              cc = c0 + 2 * p + b

              @pl.when(cc < c1)
              def _chunk(b=b, cc=cc):
                wait_in(cc, b)
                for j in range(CHUNK // 16):
                  iv = idx_raw[b, pl.ds(j * 16, 16)]
                  lid = iv - seg_base
                  valid = (lid >= 0) & (lid < SEG_TILE)
                  dummy = SEG_TILE + (lax.iota(jnp.int32, 16) & 7)
                  lid = jnp.where(valid, lid, dummy) + region0
                  idx_loc[pl.ds(j * 16, 16)] = lid
                pltpu.sync_copy(rowbuf.at[b], shared.at[idx_loc], add=True)

                @pl.when(cc + 2 < c1)
                def _next():
                  issue_in(cc + 2, b)

            return inner

          lax.fori_loop(0, lax.div(c1 - c0 + 1, 2), do_pair, 0)

          pltpu.sync_copy(
              shared.at[pl.ds(region0, SEG_TILE)],
              out_hbm.at[pl.ds(out_base, SEG_TILE), :],
          )

        return carry

      lax.fori_loop(0, max_tiles, do_tile, 0)

    return sc_pool

  B = 25000
  nb = PART_SEGS // B
  bias_vec = bias1[gene_ix]
  bias_tiled = jnp.tile(bias_vec, B // gene_n_static)[None]

  mlp = pl.pallas_call(
      _mlp_body,
      grid=(nb,),
      in_specs=[
          pl.BlockSpec((B, D), lambda i: (i, 0)),
          pl.BlockSpec((D, D), lambda i: (0, 0)),
          pl.BlockSpec((1, D), lambda i: (0, 0)),
          pl.BlockSpec((1, D), lambda i: (0, 0)),
          pl.BlockSpec((1, B), lambda i: (0, 0)),
      ],
      out_specs=pl.BlockSpec((1, 1, B), lambda i: (i, 0, 0)),
      out_shape=jax.ShapeDtypeStruct((nb, 1, B), jnp.float32),
  )

  outs = []
  for p in range(NPART):
    pooled = make_sc_pool(p * PART_TILES)(fragment_embedding, ix, bnd)
    outs.append(mlp(pooled, W1, W_direct, W2, bias_tiled).reshape(-1))

  return jnp.concatenate(outs).reshape(cell_n_static, gene_n_static)


# speedup vs baseline: 1.1531x; 1.1531x over previous
import functools

import jax
import jax.numpy as jnp
import numpy as np
from jax import lax
from jax.experimental import pallas as pl
from jax.experimental.pallas import tpu as pltpu
from jax.experimental.pallas import tpu_sc as plsc

D = 128
N_FRAG = 320000
N_SEG = 500000
SEG_TILE = 400
N_TILES = N_SEG // SEG_TILE
CHUNK = 128
REGION = 416
ZROWS = REGION // 4


def _mlp_body(s_ref, w1_ref, wd_ref, w2_ref, b_ref, o_ref):
  s = s_ref[...].astype(jnp.bfloat16)
  h = lax.dot_general(s, w1_ref[...].astype(jnp.bfloat16),
                      (((1,), (1,)), ((), ())),
                      preferred_element_type=jnp.float32)
  h = jnp.maximum(h, 0.0).astype(jnp.bfloat16)
  r = lax.dot_general(w2_ref[...].astype(jnp.bfloat16), h,
                      (((1,), (1,)), ((), ())),
                      preferred_element_type=jnp.float32)
  d = lax.dot_general(wd_ref[...].astype(jnp.bfloat16), s,
                      (((1,), (1,)), ((), ())),
                      preferred_element_type=jnp.float32)
  o_ref[...] = (r + d + b_ref[...])[None]


def kernel(fragment_embedding, fragment_cellxgene_ix, cell_n, gene_n, gene_ix,
           W_direct, W1, W2, bias1):
  gene_n_static = gene_ix.shape[0]
  cell_n_static = 5000
  zero_fold = jnp.asarray(cell_n * gene_n - cell_n_static * gene_n_static,
                          dtype=fragment_cellxgene_ix.dtype)
  ix = fragment_cellxgene_ix + zero_fold

  tile_starts = jnp.arange(N_TILES + 1, dtype=jnp.int32) * SEG_TILE
  bnd = jnp.searchsorted(ix, tile_starts, side="left")
  pad = -(N_TILES + 1) % 16 + 16
  bnd = jnp.concatenate(
      [bnd.astype(jnp.int32), jnp.zeros((pad,), jnp.int32)])

  mesh = plsc.VectorSubcoreMesh(core_axis_name="c", subcore_axis_name="s")
  info = plsc.get_sparse_core_info()
  nw = info.num_cores * info.num_subcores

  NPART = 5
  PART_TILES = N_TILES // NPART
  PART_SEGS = PART_TILES * SEG_TILE

  def make_sc_pool(tile0):
    @functools.partial(
        pl.kernel,
        out_type=jax.ShapeDtypeStruct((PART_SEGS, D), jnp.float32),
        mesh=mesh,
        scratch_types=[
            pltpu.VMEM((2, CHUNK, D), jnp.float32),
            pltpu.VMEM((CHUNK,), jnp.int32),
            pltpu.VMEM((2, CHUNK), jnp.int32),
            pltpu.VMEM((ZROWS, D), jnp.float32),
            pltpu.VMEM((N_TILES + 1 + (-(N_TILES + 1) % 16) + 16,), jnp.int32),
            pltpu.VMEM_SHARED((16 * REGION, D), jnp.float32),
            pltpu.SemaphoreType.DMA,
            pltpu.SemaphoreType.DMA,
            pltpu.SemaphoreType.DMA,
        ],
        name=f"sc_pool_{tile0}",
    )
    def sc_pool(x_hbm, ix_hbm, bnd_hbm, out_hbm, rowbuf, idx_loc, idx_raw,
                zerobuf, bndbuf, shared, sem_z, sem_b0, sem_b1):
      cid = lax.axis_index("c")
      sid = lax.axis_index("s")
      wid = cid * info.num_subcores + sid
      max_tiles = -(-PART_TILES // nw)
      region0 = sid * REGION

      def _zero_row(r, carry):
        for j in range(D // 16):
          zerobuf[r, pl.ds(j * 16, 16)] = jnp.zeros((16,), jnp.float32)
        return carry
      lax.fori_loop(0, ZROWS, _zero_row, 0)

      pltpu.sync_copy(bnd_hbm, bndbuf)

      def read_bnd(i):
        base = lax.div(i, 16) * 16
        off = i - base
        v = bndbuf[pl.ds(base, 16)]
        res = v[0]
        for k in range(1, 16):
          res = jnp.where(off == k, v[k], res)
        return res

      sems = (sem_b0, sem_b1)

      def issue_in(fs, b):
        pltpu.async_copy(x_hbm.at[pl.ds(fs, CHUNK), :], rowbuf.at[b], sems[b])
        pltpu.async_copy(ix_hbm.at[pl.ds(fs, CHUNK)], idx_raw.at[b], sems[b])

      def wait_in(fs, b):
        pltpu.make_async_copy(
            x_hbm.at[pl.ds(fs, CHUNK), :], rowbuf.at[b], sems[b]).wait()
        pltpu.make_async_copy(
            ix_hbm.at[pl.ds(fs, CHUNK)], idx_raw.at[b], sems[b]).wait()

      def do_tile(t, carry):
        gl = t * nw + wid

        @pl.when(gl < PART_TILES)
        def _():
          g = tile0 + gl
          seg_base = g * SEG_TILE
          out_base = gl * SEG_TILE
          f0 = read_bnd(g)
          f1 = read_bnd(g + 1)
          fa = f0 & jnp.int32(~7)
          nc = jnp.where(f1 > f0,
                         lax.div(f1 - fa + (CHUNK - 1), CHUNK), 0)

          def chunk_start(c):
            fs_nom = fa + c * CHUNK
            fsc = pl.multiple_of(jnp.minimum(fs_nom, N_FRAG - CHUNK), 8)
            return fs_nom, fsc

          for z in range(REGION // ZROWS):
            pltpu.async_copy(zerobuf,
                             shared.at[pl.ds(region0 + z * ZROWS, ZROWS)],
                             sem_z)
          for b in range(2):
            @pl.when(b < nc)
            def _issue(b=b):
              issue_in(chunk_start(b)[1], b)
          for z in range(REGION // ZROWS):
            pltpu.make_async_copy(
                zerobuf, shared.at[pl.ds(region0 + z * ZROWS, ZROWS)],
                sem_z).wait()

          def do_pair(p, inner):
            for b in range(2):
              cc = 2 * p + b

              @pl.when(cc < nc)
              def _chunk(b=b, cc=cc):
                fs_nom, fsc = chunk_start(cc)
                wait_in(fsc, b)
                for j in range(CHUNK // 16):
                  iv = idx_raw[b, pl.ds(j * 16, 16)]
                  pos = fsc + j * 16 + lax.iota(jnp.int32, 16)
                  lid = iv - seg_base
                  valid = (lid >= 0) & (lid < SEG_TILE) & (pos >= fs_nom)
                  dummy = SEG_TILE + (lax.iota(jnp.int32, 16) & 7)
                  lid = jnp.where(valid, lid, dummy) + region0
                  idx_loc[pl.ds(j * 16, 16)] = lid
                pltpu.sync_copy(rowbuf.at[b], shared.at[idx_loc], add=True)

                @pl.when(cc + 2 < nc)
                def _next():
                  issue_in(chunk_start(cc + 2)[1], b)

            return inner

          lax.fori_loop(0, lax.div(nc + 1, 2), do_pair, 0)

          pltpu.sync_copy(
              shared.at[pl.ds(region0, SEG_TILE)],
              out_hbm.at[pl.ds(out_base, SEG_TILE), :],
          )

        return carry

      lax.fori_loop(0, max_tiles, do_tile, 0)

    return sc_pool

  B = 20000
  nb = PART_SEGS // B
  bias_vec = bias1[gene_ix]
  bias_tiled = jnp.tile(bias_vec, B // gene_n_static)[None]

  mlp = pl.pallas_call(
      _mlp_body,
      grid=(nb,),
      in_specs=[
          pl.BlockSpec((B, D), lambda i: (i, 0)),
          pl.BlockSpec((D, D), lambda i: (0, 0)),
          pl.BlockSpec((1, D), lambda i: (0, 0)),
          pl.BlockSpec((1, D), lambda i: (0, 0)),
          pl.BlockSpec((1, B), lambda i: (0, 0)),
      ],
      out_specs=pl.BlockSpec((1, 1, B), lambda i: (i, 0, 0)),
      out_shape=jax.ShapeDtypeStruct((nb, 1, B), jnp.float32),
  )

  outs = []
  for p in range(NPART):
    pooled = make_sc_pool(p * PART_TILES)(fragment_embedding, ix, bnd)
    outs.append(mlp(pooled, W1, W_direct, W2, bias_tiled).reshape(-1))

  return jnp.concatenate(outs).reshape(cell_n_static, gene_n_static)
